# baseline (speedup 1.0000x reference)
import functools

import jax
import jax.numpy as jnp
from jax import lax
from jax.experimental import pallas as pl
from jax.experimental.pallas import tpu as pltpu
from jax.experimental.pallas import tpu_sc as plsc

_NC = 2
_NS = 16
_NW = _NC * _NS

_BATCH = 16384
_N = 200
_NCLS = 8
_HORIZON = 60
_TP = _NCLS * _NCLS * _HORIZON
_TAB = _N * _NCLS * _NCLS

_LANES = 128
_NTB = _BATCH // _LANES
_TBW = _NTB // _NW
_CN = 25
_NCH = _N // _CN
_BUF = _CN * _NCLS * _LANES

_mesh = plsc.VectorSubcoreMesh(
    core_axis_name="c", subcore_axis_name="s", num_cores=_NC, num_subcores=_NS
)


@functools.partial(
    pl.kernel,
    out_type=jax.ShapeDtypeStruct((_N, _NTB, _NCLS, _LANES), jnp.float32),
    mesh=_mesh,
    compiler_params=pltpu.CompilerParams(
        needs_layout_passes=False, use_tc_tiling_on_sc=False
    ),
    scratch_types=[
        pltpu.MemorySpace.VMEM((_TP,), jnp.float32),
        pltpu.MemorySpace.VMEM((_N,), jnp.int32),
        pltpu.MemorySpace.VMEM((_TAB,), jnp.float32),
        pltpu.MemorySpace.VMEM((_LANES * _TBW,), jnp.int32),
        pltpu.MemorySpace.VMEM((_CN, _NCLS, _LANES), jnp.float32),
        pltpu.MemorySpace.VMEM((_CN, _NCLS, _LANES), jnp.float32),
        pltpu.SemaphoreType.DMA,
        pltpu.SemaphoreType.DMA,
    ],
)
def _embed(tp_hbm, fm_hbm, cc_hbm, out_hbm, tp_v, fm_v, tab_v, cc_v, buf0,
           buf1, sem0, sem1):
    wid = lax.axis_index("s") * _NC + lax.axis_index("c")
    lanes = lax.iota(jnp.int32, 16)

    pltpu.sync_copy(tp_hbm, tp_v)
    pltpu.sync_copy(fm_hbm, fm_v)
    pltpu.sync_copy(cc_hbm.at[pl.ds(wid * _LANES * _TBW, _LANES * _TBW)], cc_v)

    @plsc.parallel_loop(0, _TAB // 16, 1, unroll=4)
    def _build(v):
        f = v * 16 + lanes
        cls = f & 7
        c = (f >> 3) & 7
        n = f >> 6
        hn = jnp.minimum(plsc.load_gather(fm_v, [n]), _HORIZON - 1)
        tab_v[pl.ds(v * 16, 16)] = plsc.load_gather(
            tp_v, [cls * (_NCLS * _HORIZON) + c * _HORIZON + hn]
        )

    def _compute(step, buf):
        t = step // _NCH
        ch = step % _NCH
        cc_vecs = [cc_v[pl.ds(t * _LANES + k * 16, 16)] for k in range(8)]

        @plsc.parallel_loop(0, _CN, 1, unroll=2)
        def _row(i):
            base = ((ch * _CN + i) * _NCLS) * _NCLS
            for c in range(_NCLS):
                for k in range(8):
                    buf[i, c, pl.ds(k * 16, 16)] = plsc.load_gather(
                        tab_v, [base + c * _NCLS + cc_vecs[k]]
                    )

    def _dst(step):
        return out_hbm.at[pl.ds((step % _NCH) * _CN, _CN),
                          wid * _TBW + step // _NCH]

    _compute(0, buf0)
    pltpu.async_copy(buf0, _dst(0), sem0)
    _compute(1, buf1)
    pltpu.async_copy(buf1, _dst(1), sem1)

    def _pair(p, _):
        for b, (buf, sem) in enumerate(((buf0, sem0), (buf1, sem1))):
            step = p * 2 + b
            pltpu.make_async_copy(buf, _dst(step), sem).wait()
            _compute(step, buf)
            pltpu.async_copy(buf, _dst(step), sem)
        return _

    lax.fori_loop(1, (_TBW * _NCH) // 2, _pair, 0)
    pltpu.make_async_copy(buf0, _dst(0), sem0).wait()
    pltpu.make_async_copy(buf1, _dst(1), sem1).wait()


def kernel(current_class, future_minutes, transition_probs):
    cc = current_class.astype(jnp.int32)
    fm = future_minutes.astype(jnp.int32)
    tp_flat = transition_probs.reshape(-1)
    out4 = _embed(tp_flat, fm, cc)
    return out4.transpose(1, 3, 0, 2).reshape(_BATCH, _N, _NCLS)


# speedup vs baseline: 1.3667x; 1.3667x over previous
import functools

import jax
import jax.numpy as jnp
from jax import lax
from jax.experimental import pallas as pl
from jax.experimental.pallas import tpu as pltpu
from jax.experimental.pallas import tpu_sc as plsc

_NC = 2
_NS = 16
_NW = _NC * _NS

_BATCH = 16384
_N = 200
_NCLS = 8
_HORIZON = 60
_TP = _NCLS * _NCLS * _HORIZON
_TAB = _N * _NCLS * _NCLS

_LANES = 128
_NTB = _BATCH // _LANES
_TBW = _NTB // _NW
_CN = 50
_NCH = _N // _CN
_BUF = _CN * _NCLS * _LANES

_mesh = plsc.VectorSubcoreMesh(
    core_axis_name="c", subcore_axis_name="s", num_cores=_NC, num_subcores=_NS
)


@functools.partial(
    pl.kernel,
    out_type=jax.ShapeDtypeStruct((_N, _NTB, _NCLS, _LANES), jnp.float32),
    mesh=_mesh,
    compiler_params=pltpu.CompilerParams(
        needs_layout_passes=False, use_tc_tiling_on_sc=False
    ),
    scratch_types=[
        pltpu.MemorySpace.VMEM((_TP,), jnp.float32),
        pltpu.MemorySpace.VMEM((_N,), jnp.int32),
        pltpu.MemorySpace.VMEM((_TAB,), jnp.float32),
        pltpu.MemorySpace.VMEM((_LANES * _TBW,), jnp.int32),
        pltpu.MemorySpace.VMEM((_CN, _NCLS, _LANES), jnp.float32),
        pltpu.MemorySpace.VMEM((_CN, _NCLS, _LANES), jnp.float32),
        pltpu.SemaphoreType.DMA,
        pltpu.SemaphoreType.DMA,
    ],
)
def _embed(tp_hbm, fm_hbm, cc_hbm, out_hbm, tp_v, fm_v, tab_v, cc_v, buf0,
           buf1, sem0, sem1):
    wid = lax.axis_index("s") * _NC + lax.axis_index("c")
    lanes = lax.iota(jnp.int32, 16)

    pltpu.sync_copy(tp_hbm, tp_v)
    pltpu.sync_copy(fm_hbm, fm_v)
    pltpu.sync_copy(cc_hbm.at[pl.ds(wid * _LANES * _TBW, _LANES * _TBW)], cc_v)

    @plsc.parallel_loop(0, _TAB // 16, 1, unroll=4)
    def _build(v):
        f = v * 16 + lanes
        cls = f & 7
        c = (f >> 3) & 7
        n = f >> 6
        hn = jnp.minimum(plsc.load_gather(fm_v, [n]), _HORIZON - 1)
        tab_v[pl.ds(v * 16, 16)] = plsc.load_gather(
            tp_v, [cls * (_NCLS * _HORIZON) + c * _HORIZON + hn]
        )

    def _compute(step, buf):
        t = step // _NCH
        ch = step % _NCH
        cc_vecs = [cc_v[pl.ds(t * _LANES + k * 16, 16)] for k in range(8)]

        @plsc.parallel_loop(0, _CN, 1, unroll=2)
        def _row(i):
            base = ((ch * _CN + i) * _NCLS) * _NCLS
            for c in range(_NCLS):
                for k in range(8):
                    buf[i, c, pl.ds(k * 16, 16)] = plsc.load_gather(
                        tab_v, [base + c * _NCLS + cc_vecs[k]]
                    )

    def _dst(step):
        return out_hbm.at[pl.ds((step % _NCH) * _CN, _CN),
                          wid * _TBW + step // _NCH]

    _compute(0, buf0)
    pltpu.async_copy(buf0, _dst(0), sem0)
    _compute(1, buf1)
    pltpu.async_copy(buf1, _dst(1), sem1)

    def _pair(p, _):
        for b, (buf, sem) in enumerate(((buf0, sem0), (buf1, sem1))):
            step = p * 2 + b
            pltpu.make_async_copy(buf, _dst(step), sem).wait()
            _compute(step, buf)
            pltpu.async_copy(buf, _dst(step), sem)
        return _

    lax.fori_loop(1, (_TBW * _NCH) // 2, _pair, 0)
    pltpu.make_async_copy(buf0, _dst(0), sem0).wait()
    pltpu.make_async_copy(buf1, _dst(1), sem1).wait()


def kernel(current_class, future_minutes, transition_probs):
    cc = current_class.astype(jnp.int32)
    fm = future_minutes.astype(jnp.int32)
    tp_flat = transition_probs.reshape(-1)
    out4 = _embed(tp_flat, fm, cc)
    return out4.transpose(1, 3, 0, 2).reshape(_BATCH, _N, _NCLS)


# speedup vs baseline: 1.3676x; 1.0007x over previous
import functools

import jax
import jax.numpy as jnp
from jax import lax
from jax.experimental import pallas as pl
from jax.experimental.pallas import tpu as pltpu
from jax.experimental.pallas import tpu_sc as plsc

_NC = 2
_NS = 16
_NW = _NC * _NS

_BATCH = 16384
_N = 200
_NCLS = 8
_HORIZON = 60
_TP = _NCLS * _NCLS * _HORIZON
_TAB = _N * _NCLS * _NCLS

_LANES = 128
_NTB = _BATCH // _LANES
_TBW = _NTB // _NW
_CN = 50
_NCH = _N // _CN

_mesh = plsc.VectorSubcoreMesh(
    core_axis_name="c", subcore_axis_name="s", num_cores=_NC, num_subcores=_NS
)


@functools.partial(
    pl.kernel,
    out_type=jax.ShapeDtypeStruct((_N, _NTB, _NCLS, _LANES), jnp.float32),
    mesh=_mesh,
    compiler_params=pltpu.CompilerParams(
        needs_layout_passes=False, use_tc_tiling_on_sc=False
    ),
    scratch_types=[
        pltpu.MemorySpace.VMEM((_TP,), jnp.float32),
        pltpu.MemorySpace.VMEM((_N,), jnp.int32),
        pltpu.MemorySpace.VMEM((_TAB,), jnp.float32),
        pltpu.MemorySpace.VMEM((_LANES * _TBW,), jnp.int32),
        pltpu.MemorySpace.VMEM((_CN, _NCLS, _LANES), jnp.float32),
        pltpu.MemorySpace.VMEM((_CN, _NCLS, _LANES), jnp.float32),
        pltpu.SemaphoreType.DMA,
        pltpu.SemaphoreType.DMA,
    ],
)
def _embed(tp_hbm, fm_hbm, cc_hbm, out_hbm, tp_v, fm_v, tab_v, cc_v, buf0,
           buf1, sem0, sem1):
    wid = lax.axis_index("s") * _NC + lax.axis_index("c")
    lanes = lax.iota(jnp.int32, 16)

    pltpu.sync_copy(tp_hbm, tp_v)
    pltpu.sync_copy(fm_hbm, fm_v)
    pltpu.sync_copy(cc_hbm.at[pl.ds(wid * _LANES * _TBW, _LANES * _TBW)], cc_v)

    @plsc.parallel_loop(0, _TAB // 16, 1, unroll=4)
    def _build(v):
        f = v * 16 + lanes
        cls = f & 7
        c = (f >> 3) & 7
        n = f >> 6
        hn = jnp.minimum(plsc.load_gather(fm_v, [n]), _HORIZON - 1)
        tab_v[pl.ds(v * 16, 16)] = plsc.load_gather(
            tp_v, [cls * (_NCLS * _HORIZON) + c * _HORIZON + hn]
        )

    def _compute(step, buf):
        t = step // _NCH
        ch = step % _NCH
        cc_vecs = [cc_v[pl.ds(t * _LANES + k * 16, 16)] for k in range(8)]

        @plsc.parallel_loop(0, _CN, 1, unroll=2)
        def _row(i):
            base = ((ch * _CN + i) * _NCLS) * _NCLS
            for c in range(_NCLS):
                for k in range(8):
                    buf[i, c, pl.ds(k * 16, 16)] = plsc.load_gather(
                        tab_v, [base + c * _NCLS + cc_vecs[k]]
                    )

    def _dst(step):
        return out_hbm.at[pl.ds((step % _NCH) * _CN, _CN),
                          wid * _TBW + step // _NCH]

    _compute(0, buf0)
    pltpu.async_copy(buf0, _dst(0), sem0)
    _compute(1, buf1)
    pltpu.async_copy(buf1, _dst(1), sem1)

    def _pair(p, _):
        for b, (buf, sem) in enumerate(((buf0, sem0), (buf1, sem1))):
            step = p * 2 + b
            pltpu.make_async_copy(buf, _dst(step), sem).wait()
            _compute(step, buf)
            pltpu.async_copy(buf, _dst(step), sem)
        return _

    lax.fori_loop(1, (_TBW * _NCH) // 2, _pair, 0)
    pltpu.make_async_copy(buf0, _dst(0), sem0).wait()
    pltpu.make_async_copy(buf1, _dst(1), sem1).wait()


def kernel(current_class, future_minutes, transition_probs):
    cc = current_class.astype(jnp.int32)
    fm = future_minutes.astype(jnp.int32)
    tp_flat = transition_probs.reshape(-1)
    out4 = _embed(tp_flat, fm, cc)
    return out4.transpose(1, 3, 0, 2).reshape(_BATCH, _N, _NCLS)
